# baseline (speedup 1.0000x reference)
import functools

import jax
import jax.numpy as jnp
from jax import lax
from jax.experimental import pallas as pl
from jax.experimental.pallas import tpu as pltpu
from jax.experimental.pallas import tpu_sc as plsc

NC = 2
NS = 16
NW = NC * NS
CHUNK = 256


@functools.partial(jax.jit, static_argnums=(3,))
def _sc_gather(idx, weight_mu, weight_logvar, b_per_w):
    B = idx.shape[0]
    D = weight_mu.shape[1]
    mesh = plsc.VectorSubcoreMesh(
        core_axis_name="c", subcore_axis_name="s",
        num_cores=NC, num_subcores=NS)

    @functools.partial(
        pl.kernel,
        out_type=[
            jax.ShapeDtypeStruct((B, D), jnp.float32),
            jax.ShapeDtypeStruct((B, D), jnp.float32),
        ],
        mesh=mesh,
        scratch_types=[
            pltpu.VMEM((b_per_w,), jnp.int32),
            pltpu.VMEM((CHUNK, D), jnp.float32),
            pltpu.SemaphoreType.DMA,
            pltpu.SemaphoreType.DMA,
        ],
    )
    def k(idx_hbm, mu_hbm, lv_hbm, out_mu, out_lv,
          idx_v, blv, sem_mu, sem_lv):
        wid = lax.axis_index("s") * NC + lax.axis_index("c")
        base = wid * b_per_w

        pltpu.sync_copy(idx_hbm.at[pl.ds(base, b_per_w)], idx_v)

        for c in range(b_per_w // CHUNK):
            cb = c * CHUNK

            def fire(g, _):
                vec = idx_v[pl.ds(cb + g * 16, 16)]
                for j in range(16):
                    i = vec[j]
                    r = g * 16 + j
                    pltpu.async_copy(mu_hbm.at[i], out_mu.at[base + cb + r],
                                     sem_mu)
                    pltpu.async_copy(lv_hbm.at[i], blv.at[r], sem_lv)
                return 0

            lax.fori_loop(0, CHUNK // 16, fire, 0)

            pltpu.make_async_copy(
                lv_hbm.at[pl.ds(0, CHUNK)], blv, sem_lv).wait()
            pltpu.sync_copy(blv, out_lv.at[pl.ds(base + cb, CHUNK)])

        pltpu.make_async_copy(
            mu_hbm.at[pl.ds(0, b_per_w)], out_mu.at[pl.ds(base, b_per_w)],
            sem_mu).wait()

    return k(idx, weight_mu, weight_logvar)


def _exp_body(lv_ref, std_ref):
    std_ref[...] = jnp.exp(0.5 * lv_ref[...])


@jax.jit
def _tc_std(logvar):
    B, D = logvar.shape
    blk = 2048
    return pl.pallas_call(
        _exp_body,
        grid=(B // blk,),
        in_specs=[pl.BlockSpec((blk, D), lambda i: (i, 0))],
        out_specs=pl.BlockSpec((blk, D), lambda i: (i, 0)),
        out_shape=jax.ShapeDtypeStruct((B, D), jnp.float32),
    )(logvar)


def kernel(idx, num_augment_pts, weight_mu, weight_logvar):
    del num_augment_pts
    B = idx.shape[0]
    assert B % NW == 0
    mu, logvar = _sc_gather(idx.astype(jnp.int32), weight_mu, weight_logvar,
                            B // NW)
    std = _tc_std(logvar)
    return (mu, mu, logvar, std)


# speedup vs baseline: 1.3310x; 1.3310x over previous
import functools

import jax
import jax.numpy as jnp
from jax import lax
from jax.experimental import pallas as pl
from jax.experimental.pallas import tpu as pltpu
from jax.experimental.pallas import tpu_sc as plsc

NC = 2
NS = 16
NW = NC * NS
LANES = 16
CHUNK = 256


@functools.partial(jax.jit, static_argnums=(3,))
def _sc_lookup(idx, weight_mu, weight_logvar, b_per_w):
    B = idx.shape[0]
    D = weight_mu.shape[1]
    mesh = plsc.VectorSubcoreMesh(
        core_axis_name="c", subcore_axis_name="s",
        num_cores=NC, num_subcores=NS)

    @functools.partial(
        pl.kernel,
        out_type=[
            jax.ShapeDtypeStruct((B, D), jnp.float32),
            jax.ShapeDtypeStruct((B, D), jnp.float32),
            jax.ShapeDtypeStruct((B, D), jnp.float32),
        ],
        mesh=mesh,
        scratch_types=[
            pltpu.VMEM((b_per_w,), jnp.int32),
            pltpu.VMEM((CHUNK, D), jnp.float32),
            pltpu.VMEM((CHUNK, D), jnp.float32),
            pltpu.VMEM((CHUNK, D), jnp.float32),
            pltpu.SemaphoreType.DMA,
            pltpu.SemaphoreType.DMA,
        ],
    )
    def k(idx_hbm, mu_hbm, lv_hbm, out_mu, out_lv, out_std,
          idx_v, bmu, blv, bstd, sem_mu, sem_lv):
        wid = lax.axis_index("s") * NC + lax.axis_index("c")
        base = wid * b_per_w

        pltpu.sync_copy(idx_hbm.at[pl.ds(base, b_per_w)], idx_v)

        n_vec = D // LANES

        for c in range(b_per_w // CHUNK):
            cb = c * CHUNK

            def fire(g, _):
                vec = idx_v[pl.ds(cb + g * 16, 16)]
                for j in range(16):
                    i = vec[j]
                    r = g * 16 + j
                    pltpu.async_copy(mu_hbm.at[i], bmu.at[r], sem_mu)
                    pltpu.async_copy(lv_hbm.at[i], blv.at[r], sem_lv)
                return 0

            lax.fori_loop(0, CHUNK // 16, fire, 0)

            pltpu.make_async_copy(
                mu_hbm.at[pl.ds(0, CHUNK)], bmu, sem_mu).wait()
            pltpu.make_async_copy(
                lv_hbm.at[pl.ds(0, CHUNK)], blv, sem_lv).wait()

            def body(t, _):
                r0 = t * 4
                for kk in range(4):
                    for j in range(n_vec):
                        v = blv[r0 + kk, pl.ds(j * LANES, LANES)]
                        bstd[r0 + kk, pl.ds(j * LANES, LANES)] = (
                            jnp.exp(0.5 * v))
                return 0

            lax.fori_loop(0, CHUNK // 4, body, 0)

            pltpu.sync_copy(bmu, out_mu.at[pl.ds(base + cb, CHUNK)])
            pltpu.sync_copy(blv, out_lv.at[pl.ds(base + cb, CHUNK)])
            pltpu.sync_copy(bstd, out_std.at[pl.ds(base + cb, CHUNK)])

    return k(idx, weight_mu, weight_logvar)


def kernel(idx, num_augment_pts, weight_mu, weight_logvar):
    del num_augment_pts
    B = idx.shape[0]
    assert B % NW == 0
    mu, logvar, std = _sc_lookup(idx.astype(jnp.int32), weight_mu,
                                 weight_logvar, B // NW)
    return (mu, mu, logvar, std)
